# baseline (speedup 1.0000x reference)
import functools

import jax
import jax.numpy as jnp
from jax import lax
from jax.experimental import pallas as pl
from jax.experimental.pallas import tpu as pltpu
from jax.experimental.pallas import tpu_sc as plsc

N = 10000
E = 320000
D = 128
H = 128
OUT = 128
L = 3
G = 128

NW = 32
NP = 10112
NCHUNK = 80
NGRP = 5
CPG = 32
EP = NCHUNK * 128
EPAD = NW * EP
RPT = NP // 16
GPW = G // NW
CR = 64
F32 = jnp.float32
BF16 = jnp.bfloat16
NEG = -3.0e38

_mesh = plsc.VectorSubcoreMesh(core_axis_name="c", subcore_axis_name="s")
_sc_params = pltpu.CompilerParams(use_tc_tiling_on_sc=False)


def _deg_body(dst_hbm, deg_out, dst_v, ones_v, z16, dacc):
    c = lax.axis_index("c")
    s = lax.axis_index("s")
    wid = s * 2 + c

    zero16 = jnp.zeros((16,), F32)
    one16 = jnp.ones((16,), F32)

    def zsmall(i, _):
        ones_v[i, :] = one16
        return 0

    lax.fori_loop(0, 128, zsmall, 0)

    def z16f(i, _):
        z16[i, :] = zero16
        return 0

    lax.fori_loop(0, 160, z16f, 0)

    base = s * RPT
    for k in range(3):
        pltpu.sync_copy(z16, dacc.at[pl.ds(base + k * 160, 160)])
    pltpu.sync_copy(z16.at[pl.ds(0, RPT - 480)],
                    dacc.at[pl.ds(base + 480, RPT - 480)])

    plsc.subcore_barrier()

    def grp(g, _):
        pltpu.sync_copy(dst_hbm.at[wid, pl.ds(g * 16, 16)], dst_v)

        def edge_chunk(j, _):
            pltpu.sync_copy(ones_v, dacc.at[dst_v.at[j]], add=True)
            return 0

        lax.fori_loop(0, 16, edge_chunk, 0)
        return 0

    lax.fori_loop(0, NGRP, grp, 0)

    plsc.subcore_barrier()

    pltpu.sync_copy(dacc.at[pl.ds(base, RPT)],
                    deg_out.at[c, pl.ds(base, RPT)])


_sc_deg = pl.kernel(
    _deg_body,
    out_type=(jax.ShapeDtypeStruct((2, NP, 16), F32),),
    mesh=_mesh,
    scratch_types=(
        pltpu.VMEM((16, 128), jnp.int32),
        pltpu.VMEM((128, 16), F32),
        pltpu.VMEM((160, 16), F32),
        pltpu.VMEM_SHARED((NP, 16), F32),
    ),
    compiler_params=_sc_params,
    name="sc_deg",
)


def _readout(h_hbm, batch_hbm, rosum_out, romax_out, rocnt_out, batch_v,
             hbuf, osum, omax, ocnt, wid):
    zero16 = jnp.zeros((16,), F32)
    one16 = jnp.ones((16,), F32)
    g0 = wid * GPW

    cnts = tuple(jnp.zeros((16,), F32) for _ in range(GPW + 1))
    for bc in range(5):
        size = 2048 if bc < 4 else NP - 4 * 2048
        pltpu.sync_copy(batch_hbm.at[pl.ds(bc * 2048, size)],
                        batch_v.at[pl.ds(0, size)])

        def cnt_body(j, carry):
            v = batch_v[pl.ds(j * 16, 16)]
            return tuple(
                carry[t] + jnp.where(v < (g0 + t), one16, zero16)
                for t in range(GPW + 1)
            )

        cnts = lax.fori_loop(0, size // 16, cnt_body, cnts)
    starts = []
    for t in range(GPW + 1):
        v = cnts[t]
        tot = v[0]
        for i in range(1, 16):
            tot = tot + v[i]
        starts.append(tot.astype(jnp.int32))

    neg16 = jnp.full((16,), NEG, dtype=F32)

    for gl in range(GPW):
        sg = starts[gl]
        eg = starts[gl + 1]
        k0 = sg // CR
        k1 = (eg + (CR - 1)) // CR

        def chunk_body(k, carry, sg=sg, eg=eg):
            pltpu.sync_copy(h_hbm.at[pl.ds(k * CR, CR)], hbuf)

            def row_body(r, carry2):
                sums, maxs = carry2
                gi = k * CR + r
                valid = jnp.logical_and(gi >= sg, gi < eg)
                nsums = []
                nmaxs = []
                for dd in range(8):
                    v = hbuf[r, pl.ds(dd * 16, 16)]
                    nsums.append(sums[dd] + jnp.where(valid, v, zero16))
                    nmaxs.append(jnp.maximum(maxs[dd],
                                             jnp.where(valid, v, neg16)))
                return (tuple(nsums), tuple(nmaxs))

            return lax.fori_loop(0, CR, row_body, carry)

        acc_init = (tuple(jnp.zeros((16,), F32) for _ in range(8)),
                    tuple(jnp.full((16,), NEG, dtype=F32) for _ in range(8)))
        sums, maxs = lax.fori_loop(k0, k1, chunk_body, acc_init)

        cntf = (eg - sg).astype(F32)
        nonempty = eg > sg
        for dd in range(8):
            osum[gl, pl.ds(dd * 16, 16)] = sums[dd]
            omax[gl, pl.ds(dd * 16, 16)] = jnp.where(nonempty, maxs[dd],
                                                     zero16)
        ocnt[gl, :] = one16 * cntf

    pltpu.sync_copy(osum, rosum_out.at[wid])
    pltpu.sync_copy(omax, romax_out.at[wid])
    pltpu.sync_copy(ocnt, rocnt_out.at[wid])


def _msg_body(hbf_hbm, h32_hbm, src_hbm, dst_hbm, batch_hbm, msg_out,
              rosum_out, romax_out, rocnt_out, batch_v, hbuf, osum, omax,
              ocnt, src_v, dst_v, bf0, bf1, bf2, bf3, f0, f1, acc,
              sg0, sg1, sg2, sg3):
    c = lax.axis_index("c")
    s = lax.axis_index("s")
    wid = s * 2 + c

    zero16 = jnp.zeros((16,), F32)

    def zf(i, _):
        for dd in range(8):
            f0[i, pl.ds(dd * 16, 16)] = zero16
        return 0

    lax.fori_loop(0, 64, zf, 0)

    base = s * RPT
    for k in range(9):
        pltpu.sync_copy(f0, acc.at[pl.ds(base + k * 64, 64)])
    pltpu.sync_copy(f0.at[pl.ds(0, RPT - 576)],
                    acc.at[pl.ds(base + 576, RPT - 576)])

    plsc.subcore_barrier()

    bfs = [bf0, bf1, bf2, bf3]
    gsems = [sg0, sg1, sg2, sg3]
    fs = [f0, f1]

    def grp(g, _):
        pltpu.sync_copy(src_hbm.at[wid, g], src_v)
        pltpu.sync_copy(dst_hbm.at[wid, g], dst_v)
        for t in range(4):
            pltpu.make_async_copy(hbf_hbm.at[src_v.at[t]],
                                  bfs[t], gsems[t]).start()

        def quad(q, _):
            for t in range(4):
                j = q * 4 + t
                pltpu.make_async_copy(hbf_hbm.at[src_v.at[j]],
                                      bfs[t], gsems[t]).wait()

                fb = fs[t % 2]

                def conv(r, _, bt=bfs[t], fb=fb):
                    for dd in range(4):
                        vi = bt[r, pl.ds(dd * 16, 16)]
                        va = lax.bitcast_convert_type(vi << 16, F32)
                        vb = lax.bitcast_convert_type(vi & jnp.int32(-65536),
                                                      F32)
                        fb[r, pl.ds(dd * 16, 16)] = va
                        fb[r, pl.ds(64 + dd * 16, 16)] = vb
                    return 0

                lax.fori_loop(0, 64, conv, 0)
                pltpu.sync_copy(fb, acc.at[dst_v.at[j]], add=True)

                @pl.when(q < (CPG // 4) - 1)
                def _():
                    pltpu.make_async_copy(hbf_hbm.at[src_v.at[j + 4]],
                                          bfs[t], gsems[t]).start()
            return 0

        lax.fori_loop(0, CPG // 4, quad, 0)
        return 0

    lax.fori_loop(0, NGRP, grp, 0)

    plsc.subcore_barrier()

    pltpu.sync_copy(acc.at[pl.ds(base, RPT)],
                    msg_out.at[c, pl.ds(base, RPT)])

    _readout(h32_hbm, batch_hbm, rosum_out, romax_out, rocnt_out,
             batch_v, hbuf, osum, omax, ocnt, wid)


_RO_OUT = (
    jax.ShapeDtypeStruct((NW, GPW, D), F32),
    jax.ShapeDtypeStruct((NW, GPW, D), F32),
    jax.ShapeDtypeStruct((NW, GPW, 16), F32),
)
_RO_SCRATCH = (
    pltpu.VMEM((2048,), jnp.int32),
    pltpu.VMEM((CR, D), F32),
    pltpu.VMEM((GPW, D), F32),
    pltpu.VMEM((GPW, D), F32),
    pltpu.VMEM((GPW, 16), F32),
)

_sc_msg = pl.kernel(
    _msg_body,
    out_type=(jax.ShapeDtypeStruct((2, NP, D), F32),) + _RO_OUT,
    mesh=_mesh,
    scratch_types=_RO_SCRATCH + (
        pltpu.VMEM((CPG, 64), jnp.int32),
        pltpu.VMEM((CPG, 64), jnp.int32),
        pltpu.VMEM((64, 64), jnp.int32),
        pltpu.VMEM((64, 64), jnp.int32),
        pltpu.VMEM((64, 64), jnp.int32),
        pltpu.VMEM((64, 64), jnp.int32),
        pltpu.VMEM((64, D), F32),
        pltpu.VMEM((64, D), F32),
        pltpu.VMEM_SHARED((NP, D), F32),
        pltpu.SemaphoreType.DMA,
        pltpu.SemaphoreType.DMA,
        pltpu.SemaphoreType.DMA,
        pltpu.SemaphoreType.DMA,
    ),
    compiler_params=_sc_params,
    name="sc_msg_readout",
)


def _ro_body(h32_hbm, batch_hbm, rosum_out, romax_out, rocnt_out,
             batch_v, hbuf, osum, omax, ocnt):
    c = lax.axis_index("c")
    s = lax.axis_index("s")
    wid = s * 2 + c
    _readout(h32_hbm, batch_hbm, rosum_out, romax_out, rocnt_out,
             batch_v, hbuf, osum, omax, ocnt, wid)


_sc_ro = pl.kernel(
    _ro_body,
    out_type=_RO_OUT,
    mesh=_mesh,
    scratch_types=_RO_SCRATCH,
    compiler_params=_sc_params,
    name="sc_readout",
)


def _enc_body(x_ref, w_ref, b_ref, o_ref):
    h = jnp.dot(x_ref[:], w_ref[:], preferred_element_type=F32) + b_ref[:]
    o_ref[pl.ds(0, N), :] = h
    o_ref[pl.ds(N, NP - N), :] = jnp.zeros((NP - N, D), F32)


_enc_call = pl.pallas_call(
    _enc_body,
    out_shape=jax.ShapeDtypeStruct((NP, D), F32),
)


def _softmax_row(a):
    m = jnp.max(a, axis=1, keepdims=True)
    e = jnp.exp(a - m)
    return e / jnp.sum(e, axis=1, keepdims=True)


def _dense_body(h_ref, p_ref, deg_ref, w0_ref, w1_ref, w2_ref, b0_ref,
                b1_ref, b2_ref, gam_ref, bet_ref, ana_ref, aact_ref, o_ref):
    deg = deg_ref[0, :, 0:1] + deg_ref[1, :, 0:1]
    inv = 1.0 / jnp.maximum(deg, 1.0)
    msum = p_ref[0] + p_ref[1]
    mmean = msum * inv
    h = h_ref[:]
    gcn = jnp.dot(mmean, w0_ref[:], preferred_element_type=F32) + b0_ref[:]
    sage = jnp.dot(h + mmean, w1_ref[:], preferred_element_type=F32) + b1_ref[:]
    gin = jnp.dot(h + msum, w2_ref[:], preferred_element_type=F32) + b2_ref[:]
    wna = _softmax_row(ana_ref[:])
    t = wna[0, 0] * gcn + wna[0, 1] * sage + wna[0, 2] * gin
    mask = lax.broadcasted_iota(jnp.int32, (NP, 1), 0) < N
    tm = jnp.where(mask, t, 0.0)
    mu = jnp.sum(tm, axis=0, keepdims=True) * (1.0 / N)
    dv = jnp.where(mask, t - mu, 0.0)
    var = jnp.sum(dv * dv, axis=0, keepdims=True) * (1.0 / N)
    hn = (t - mu) * lax.rsqrt(var + 1e-5) * gam_ref[:] + bet_ref[:]
    wact = _softmax_row(aact_ref[:])
    act = (wact[0, 0] * jnp.maximum(hn, 0.0)
           + wact[0, 1] * jnp.tanh(hn)
           + wact[0, 2] * jnp.where(hn > 0, hn, jnp.exp(hn) - 1.0))
    o_ref[:] = jnp.where(mask, act, 0.0)


_dense_call = pl.pallas_call(
    _dense_body,
    out_shape=jax.ShapeDtypeStruct((NP, D), F32),
)


def _final_body(rs0, rs1, rs2, rs3, rm0, rm1, rm2, rm3, cnt_ref, aro_ref,
                wc_ref, bc_ref, o_ref):
    cnt = jnp.maximum(cnt_ref[:, 0:1], 1.0)
    w = _softmax_row(aro_ref[:])
    rep = jnp.zeros((G, D), F32)
    for i, (rs, rm) in enumerate([(rs0, rm0), (rs1, rm1), (rs2, rm2),
                                  (rs3, rm3)]):
        sm = rs[:]
        rep = rep + (w[i, 0] * (sm / cnt) + w[i, 1] * sm + w[i, 2] * rm[:])
    o_ref[:] = jnp.dot(rep, wc_ref[:], preferred_element_type=F32) + bc_ref[:]


_final_call = pl.pallas_call(
    _final_body,
    out_shape=jax.ShapeDtypeStruct((G, OUT), F32),
)


def _swz(h):
    pairs = jnp.stack([h[:, :64], h[:, 64:]], axis=-1).astype(BF16)
    return jax.lax.bitcast_convert_type(pairs, jnp.int32)


def kernel(x, edge_index, batch, W_enc, b_enc, W_ops, b_ops, bn_gamma,
           bn_beta, alpha_na, alpha_act, alpha_ro, W_cls, b_cls):
    pad = EPAD - E
    src_p = jnp.concatenate([edge_index[0], jnp.zeros((pad,), jnp.int32)])
    dst_p = jnp.concatenate([edge_index[1], jnp.full((pad,), N, jnp.int32)])
    dst_r = dst_p.reshape(NW, NCHUNK, 128)
    src_r4 = src_p.reshape(NW, NGRP, CPG, 64)
    dst_r4 = dst_p.reshape(NW, NGRP, CPG, 64)
    batch_p = jnp.concatenate([batch, jnp.full((NP - N,), G, jnp.int32)])

    def padrow(a):
        return jnp.concatenate(
            [a, jnp.full((a.shape[0], 128 - a.shape[1]), -1e30, F32)], axis=1)

    ana_p = padrow(alpha_na)
    aact_p = padrow(alpha_act)
    aro_p = padrow(alpha_ro)

    (deg2,) = _sc_deg(dst_r)
    h = _enc_call(x, W_enc, b_enc.reshape(1, H))

    ros, rom = [], []
    roc = None
    for l in range(L):
        p, rs, rm, rc = _sc_msg(_swz(h), h, src_r4, dst_r4, batch_p)
        if l == 0:
            roc = rc.reshape(G, 16)
        ros.append(rs.reshape(G, D))
        rom.append(rm.reshape(G, D))
        h = _dense_call(
            h, p, deg2,
            W_ops[l, 0], W_ops[l, 1], W_ops[l, 2],
            b_ops[l, 0].reshape(1, H), b_ops[l, 1].reshape(1, H),
            b_ops[l, 2].reshape(1, H),
            bn_gamma[l].reshape(1, H), bn_beta[l].reshape(1, H),
            ana_p[l:l + 1], aact_p[l:l + 1])
    rs, rm, _ = _sc_ro(h, batch_p)
    ros.append(rs.reshape(G, D))
    rom.append(rm.reshape(G, D))

    return _final_call(ros[0], ros[1], ros[2], ros[3], rom[0], rom[1],
                       rom[2], rom[3], roc, aro_p, W_cls,
                       b_cls.reshape(1, OUT))


# speedup vs baseline: 1.0650x; 1.0650x over previous
import functools

import jax
import jax.numpy as jnp
from jax import lax
from jax.experimental import pallas as pl
from jax.experimental.pallas import tpu as pltpu
from jax.experimental.pallas import tpu_sc as plsc

N = 10000
E = 320000
D = 128
H = 128
OUT = 128
L = 3
G = 128

NW = 32
NP = 10112
NCHUNK = 80
NGRP = 5
CPG = 32
EP = NCHUNK * 128
EPAD = NW * EP
RPT = NP // 16
GPW = G // NW
CR = 64
F32 = jnp.float32
BF16 = jnp.bfloat16
NEG = -3.0e38

_mesh = plsc.VectorSubcoreMesh(core_axis_name="c", subcore_axis_name="s")
_sc_params = pltpu.CompilerParams(use_tc_tiling_on_sc=False)


def _deg_body(dst_hbm, deg_out, dst_v, ones_v, z16, dacc):
    c = lax.axis_index("c")
    s = lax.axis_index("s")
    wid = s * 2 + c

    zero16 = jnp.zeros((16,), F32)
    one16 = jnp.ones((16,), F32)

    def zsmall(i, _):
        ones_v[i, :] = one16
        return 0

    lax.fori_loop(0, 128, zsmall, 0)

    def z16f(i, _):
        z16[i, :] = zero16
        return 0

    lax.fori_loop(0, 160, z16f, 0)

    base = s * RPT
    for k in range(3):
        pltpu.sync_copy(z16, dacc.at[pl.ds(base + k * 160, 160)])
    pltpu.sync_copy(z16.at[pl.ds(0, RPT - 480)],
                    dacc.at[pl.ds(base + 480, RPT - 480)])

    plsc.subcore_barrier()

    def grp(g, _):
        pltpu.sync_copy(dst_hbm.at[wid, pl.ds(g * 16, 16)], dst_v)

        def edge_chunk(j, _):
            pltpu.sync_copy(ones_v, dacc.at[dst_v.at[j]], add=True)
            return 0

        lax.fori_loop(0, 16, edge_chunk, 0)
        return 0

    lax.fori_loop(0, NGRP, grp, 0)

    plsc.subcore_barrier()

    pltpu.sync_copy(dacc.at[pl.ds(base, RPT)],
                    deg_out.at[c, pl.ds(base, RPT)])


_sc_deg = pl.kernel(
    _deg_body,
    out_type=(jax.ShapeDtypeStruct((2, NP, 16), F32),),
    mesh=_mesh,
    scratch_types=(
        pltpu.VMEM((16, 128), jnp.int32),
        pltpu.VMEM((128, 16), F32),
        pltpu.VMEM((160, 16), F32),
        pltpu.VMEM_SHARED((NP, 16), F32),
    ),
    compiler_params=_sc_params,
    name="sc_deg",
)


def _readout(h_hbm, batch_hbm, rosum_out, romax_out, rocnt_out, batch_v,
             hbuf, osum, omax, ocnt, wid):
    zero16 = jnp.zeros((16,), F32)
    one16 = jnp.ones((16,), F32)
    g0 = wid * GPW

    cnts = tuple(jnp.zeros((16,), F32) for _ in range(GPW + 1))
    for bc in range(5):
        size = 2048 if bc < 4 else NP - 4 * 2048
        pltpu.sync_copy(batch_hbm.at[pl.ds(bc * 2048, size)],
                        batch_v.at[pl.ds(0, size)])

        def cnt_body(j, carry):
            v = batch_v[pl.ds(j * 16, 16)]
            return tuple(
                carry[t] + jnp.where(v < (g0 + t), one16, zero16)
                for t in range(GPW + 1)
            )

        cnts = lax.fori_loop(0, size // 16, cnt_body, cnts)
    starts = []
    for t in range(GPW + 1):
        v = cnts[t]
        tot = v[0]
        for i in range(1, 16):
            tot = tot + v[i]
        starts.append(tot.astype(jnp.int32))

    neg16 = jnp.full((16,), NEG, dtype=F32)

    for gl in range(GPW):
        sg = starts[gl]
        eg = starts[gl + 1]
        k0 = sg // CR
        k1 = (eg + (CR - 1)) // CR

        def chunk_body(k, carry, sg=sg, eg=eg):
            pltpu.sync_copy(h_hbm.at[pl.ds(k * CR, CR)], hbuf)

            def row_body(r, carry2):
                sums, maxs = carry2
                gi = k * CR + r
                valid = jnp.logical_and(gi >= sg, gi < eg)
                nsums = []
                nmaxs = []
                for dd in range(8):
                    v = hbuf[r, pl.ds(dd * 16, 16)]
                    nsums.append(sums[dd] + jnp.where(valid, v, zero16))
                    nmaxs.append(jnp.maximum(maxs[dd],
                                             jnp.where(valid, v, neg16)))
                return (tuple(nsums), tuple(nmaxs))

            return lax.fori_loop(0, CR, row_body, carry)

        acc_init = (tuple(jnp.zeros((16,), F32) for _ in range(8)),
                    tuple(jnp.full((16,), NEG, dtype=F32) for _ in range(8)))
        sums, maxs = lax.fori_loop(k0, k1, chunk_body, acc_init)

        cntf = (eg - sg).astype(F32)
        nonempty = eg > sg
        for dd in range(8):
            osum[gl, pl.ds(dd * 16, 16)] = sums[dd]
            omax[gl, pl.ds(dd * 16, 16)] = jnp.where(nonempty, maxs[dd],
                                                     zero16)
        ocnt[gl, :] = one16 * cntf

    pltpu.sync_copy(osum, rosum_out.at[wid])
    pltpu.sync_copy(omax, romax_out.at[wid])
    pltpu.sync_copy(ocnt, rocnt_out.at[wid])


def _msg_body(hbf_hbm, h32_hbm, src_hbm, dst_hbm, batch_hbm, msg_out,
              rosum_out, romax_out, rocnt_out, batch_v, hbuf, osum, omax,
              ocnt, src_v, dst_v, bf0, bf1, bf2, bf3, f0, f1, acc,
              sg0, sg1, sg2, sg3, ss0, ss1):
    c = lax.axis_index("c")
    s = lax.axis_index("s")
    wid = s * 2 + c

    zero16 = jnp.zeros((16,), F32)

    def zf(i, _):
        for dd in range(8):
            f0[i, pl.ds(dd * 16, 16)] = zero16
        return 0

    lax.fori_loop(0, 64, zf, 0)

    base = s * RPT
    for k in range(9):
        pltpu.sync_copy(f0, acc.at[pl.ds(base + k * 64, 64)])
    pltpu.sync_copy(f0.at[pl.ds(0, RPT - 576)],
                    acc.at[pl.ds(base + 576, RPT - 576)])

    plsc.subcore_barrier()

    bfs = [bf0, bf1, bf2, bf3]
    gsems = [sg0, sg1, sg2, sg3]
    fs = [f0, f1]
    ssems = [ss0, ss1]

    def grp(g, _):
        pltpu.sync_copy(src_hbm.at[wid, g], src_v)
        pltpu.sync_copy(dst_hbm.at[wid, g], dst_v)
        for t in range(4):
            pltpu.make_async_copy(hbf_hbm.at[src_v.at[t]],
                                  bfs[t], gsems[t]).start()

        def quad(q, _):
            scats = []
            for t in range(4):
                j = q * 4 + t
                pltpu.make_async_copy(hbf_hbm.at[src_v.at[j]],
                                      bfs[t], gsems[t]).wait()

                fb = fs[t % 2]
                if t >= 2:
                    scats[t - 2].wait()

                def conv(r2, _, bt=bfs[t], fb=fb):
                    for u in range(2):
                        r = r2 * 2 + u
                        for dd in range(4):
                            vi = bt[r, pl.ds(dd * 16, 16)]
                            va = lax.bitcast_convert_type(vi << 16, F32)
                            vb = lax.bitcast_convert_type(
                                vi & jnp.int32(-65536), F32)
                            fb[r, pl.ds(dd * 16, 16)] = va
                            fb[r, pl.ds(64 + dd * 16, 16)] = vb
                    return 0

                lax.fori_loop(0, 32, conv, 0)

                @pl.when(q < (CPG // 4) - 1)
                def _():
                    pltpu.make_async_copy(hbf_hbm.at[src_v.at[j + 4]],
                                          bfs[t], gsems[t]).start()

                scats.append(pltpu.async_copy(fb, acc.at[dst_v.at[j]],
                                              ssems[t % 2], add=True))
            scats[2].wait()
            scats[3].wait()
            return 0

        lax.fori_loop(0, CPG // 4, quad, 0)
        return 0

    lax.fori_loop(0, NGRP, grp, 0)

    plsc.subcore_barrier()

    pltpu.sync_copy(acc.at[pl.ds(base, RPT)],
                    msg_out.at[c, pl.ds(base, RPT)])

    _readout(h32_hbm, batch_hbm, rosum_out, romax_out, rocnt_out,
             batch_v, hbuf, osum, omax, ocnt, wid)


_RO_OUT = (
    jax.ShapeDtypeStruct((NW, GPW, D), F32),
    jax.ShapeDtypeStruct((NW, GPW, D), F32),
    jax.ShapeDtypeStruct((NW, GPW, 16), F32),
)
_RO_SCRATCH = (
    pltpu.VMEM((2048,), jnp.int32),
    pltpu.VMEM((CR, D), F32),
    pltpu.VMEM((GPW, D), F32),
    pltpu.VMEM((GPW, D), F32),
    pltpu.VMEM((GPW, 16), F32),
)

_sc_msg = pl.kernel(
    _msg_body,
    out_type=(jax.ShapeDtypeStruct((2, NP, D), F32),) + _RO_OUT,
    mesh=_mesh,
    scratch_types=_RO_SCRATCH + (
        pltpu.VMEM((CPG, 64), jnp.int32),
        pltpu.VMEM((CPG, 64), jnp.int32),
        pltpu.VMEM((64, 64), jnp.int32),
        pltpu.VMEM((64, 64), jnp.int32),
        pltpu.VMEM((64, 64), jnp.int32),
        pltpu.VMEM((64, 64), jnp.int32),
        pltpu.VMEM((64, D), F32),
        pltpu.VMEM((64, D), F32),
        pltpu.VMEM_SHARED((NP, D), F32),
        pltpu.SemaphoreType.DMA,
        pltpu.SemaphoreType.DMA,
        pltpu.SemaphoreType.DMA,
        pltpu.SemaphoreType.DMA,
        pltpu.SemaphoreType.DMA,
        pltpu.SemaphoreType.DMA,
    ),
    compiler_params=_sc_params,
    name="sc_msg_readout",
)


def _ro_body(h32_hbm, batch_hbm, rosum_out, romax_out, rocnt_out,
             batch_v, hbuf, osum, omax, ocnt):
    c = lax.axis_index("c")
    s = lax.axis_index("s")
    wid = s * 2 + c
    _readout(h32_hbm, batch_hbm, rosum_out, romax_out, rocnt_out,
             batch_v, hbuf, osum, omax, ocnt, wid)


_sc_ro = pl.kernel(
    _ro_body,
    out_type=_RO_OUT,
    mesh=_mesh,
    scratch_types=_RO_SCRATCH,
    compiler_params=_sc_params,
    name="sc_readout",
)


def _enc_body(x_ref, w_ref, b_ref, o_ref):
    h = jnp.dot(x_ref[:], w_ref[:], preferred_element_type=F32) + b_ref[:]
    o_ref[pl.ds(0, N), :] = h
    o_ref[pl.ds(N, NP - N), :] = jnp.zeros((NP - N, D), F32)


_enc_call = pl.pallas_call(
    _enc_body,
    out_shape=jax.ShapeDtypeStruct((NP, D), F32),
)


def _softmax_row(a):
    m = jnp.max(a, axis=1, keepdims=True)
    e = jnp.exp(a - m)
    return e / jnp.sum(e, axis=1, keepdims=True)


def _dense_body(h_ref, p_ref, deg_ref, w0_ref, w1_ref, w2_ref, b0_ref,
                b1_ref, b2_ref, gam_ref, bet_ref, ana_ref, aact_ref, o_ref):
    deg = deg_ref[0, :, 0:1] + deg_ref[1, :, 0:1]
    inv = 1.0 / jnp.maximum(deg, 1.0)
    msum = p_ref[0] + p_ref[1]
    mmean = msum * inv
    h = h_ref[:]
    gcn = jnp.dot(mmean, w0_ref[:], preferred_element_type=F32) + b0_ref[:]
    sage = jnp.dot(h + mmean, w1_ref[:], preferred_element_type=F32) + b1_ref[:]
    gin = jnp.dot(h + msum, w2_ref[:], preferred_element_type=F32) + b2_ref[:]
    wna = _softmax_row(ana_ref[:])
    t = wna[0, 0] * gcn + wna[0, 1] * sage + wna[0, 2] * gin
    mask = lax.broadcasted_iota(jnp.int32, (NP, 1), 0) < N
    tm = jnp.where(mask, t, 0.0)
    mu = jnp.sum(tm, axis=0, keepdims=True) * (1.0 / N)
    dv = jnp.where(mask, t - mu, 0.0)
    var = jnp.sum(dv * dv, axis=0, keepdims=True) * (1.0 / N)
    hn = (t - mu) * lax.rsqrt(var + 1e-5) * gam_ref[:] + bet_ref[:]
    wact = _softmax_row(aact_ref[:])
    act = (wact[0, 0] * jnp.maximum(hn, 0.0)
           + wact[0, 1] * jnp.tanh(hn)
           + wact[0, 2] * jnp.where(hn > 0, hn, jnp.exp(hn) - 1.0))
    o_ref[:] = jnp.where(mask, act, 0.0)


_dense_call = pl.pallas_call(
    _dense_body,
    out_shape=jax.ShapeDtypeStruct((NP, D), F32),
)


def _final_body(rs0, rs1, rs2, rs3, rm0, rm1, rm2, rm3, cnt_ref, aro_ref,
                wc_ref, bc_ref, o_ref):
    cnt = jnp.maximum(cnt_ref[:, 0:1], 1.0)
    w = _softmax_row(aro_ref[:])
    rep = jnp.zeros((G, D), F32)
    for i, (rs, rm) in enumerate([(rs0, rm0), (rs1, rm1), (rs2, rm2),
                                  (rs3, rm3)]):
        sm = rs[:]
        rep = rep + (w[i, 0] * (sm / cnt) + w[i, 1] * sm + w[i, 2] * rm[:])
    o_ref[:] = jnp.dot(rep, wc_ref[:], preferred_element_type=F32) + bc_ref[:]


_final_call = pl.pallas_call(
    _final_body,
    out_shape=jax.ShapeDtypeStruct((G, OUT), F32),
)


def _swz(h):
    pairs = jnp.stack([h[:, :64], h[:, 64:]], axis=-1).astype(BF16)
    return jax.lax.bitcast_convert_type(pairs, jnp.int32)


def kernel(x, edge_index, batch, W_enc, b_enc, W_ops, b_ops, bn_gamma,
           bn_beta, alpha_na, alpha_act, alpha_ro, W_cls, b_cls):
    pad = EPAD - E
    src_p = jnp.concatenate([edge_index[0], jnp.zeros((pad,), jnp.int32)])
    dst_p = jnp.concatenate([edge_index[1], jnp.full((pad,), N, jnp.int32)])
    dst_r = dst_p.reshape(NW, NCHUNK, 128)
    src_r4 = src_p.reshape(NW, NGRP, CPG, 64)
    dst_r4 = dst_p.reshape(NW, NGRP, CPG, 64)
    batch_p = jnp.concatenate([batch, jnp.full((NP - N,), G, jnp.int32)])

    def padrow(a):
        return jnp.concatenate(
            [a, jnp.full((a.shape[0], 128 - a.shape[1]), -1e30, F32)], axis=1)

    ana_p = padrow(alpha_na)
    aact_p = padrow(alpha_act)
    aro_p = padrow(alpha_ro)

    (deg2,) = _sc_deg(dst_r)
    h = _enc_call(x, W_enc, b_enc.reshape(1, H))

    ros, rom = [], []
    roc = None
    for l in range(L):
        p, rs, rm, rc = _sc_msg(_swz(h), h, src_r4, dst_r4, batch_p)
        if l == 0:
            roc = rc.reshape(G, 16)
        ros.append(rs.reshape(G, D))
        rom.append(rm.reshape(G, D))
        h = _dense_call(
            h, p, deg2,
            W_ops[l, 0], W_ops[l, 1], W_ops[l, 2],
            b_ops[l, 0].reshape(1, H), b_ops[l, 1].reshape(1, H),
            b_ops[l, 2].reshape(1, H),
            bn_gamma[l].reshape(1, H), bn_beta[l].reshape(1, H),
            ana_p[l:l + 1], aact_p[l:l + 1])
    rs, rm, _ = _sc_ro(h, batch_p)
    ros.append(rs.reshape(G, D))
    rom.append(rm.reshape(G, D))

    return _final_call(ros[0], ros[1], ros[2], ros[3], rom[0], rom[1],
                       rom[2], rom[3], roc, aro_p, W_cls,
                       b_cls.reshape(1, OUT))
